# baseline (speedup 1.0000x reference)
import functools

import jax
import jax.numpy as jnp
from jax import lax
from jax.experimental import pallas as pl
from jax.experimental.pallas import tpu as pltpu
from jax.experimental.pallas import tpu_sc as plsc

VOCAB = 1_000_000
HIDDEN = 32
BATCH = 4096
HIST = 50
NC, NS, LANES = 2, 16, 16
NW = NC * NS
BPW = BATCH // NW
RPB = 128 // HIDDEN
NGRP = BPW // LANES


def _sc_body(ctxT_hbm, t_hbm, in_emb_hbm, out_emb_hbm, conT_hbm, tgtT_hbm,
             ctx_v, tidx_v, big_v, off_v, tbig_v, toff_v,
             rows0, rows1, rowst, accT, tgtT_v, sem0, sem1, semt):
    wid = lax.axis_index("s") * NC + lax.axis_index("c")
    base = wid * BPW

    pltpu.sync_copy(ctxT_hbm.at[:, pl.ds(base, BPW)], ctx_v)
    pltpu.sync_copy(t_hbm.at[pl.ds(base, BPW)], tidx_v)

    @pl.loop(0, BPW // LANES)
    def _tprep(c):
        v = tidx_v[pl.ds(c * LANES, LANES)]
        tbig_v[pl.ds(c * LANES, LANES)] = lax.shift_right_logical(v, 2)
        toff_v[pl.ds(c * LANES, LANES)] = lax.shift_left(
            lax.bitwise_and(v, 3), 5)

    pltpu.async_copy(out_emb_hbm.at[tbig_v], rowst, semt)

    @pl.loop(0, HIST)
    def _cprep(h):
        for c in range(NGRP):
            v = ctx_v[h, pl.ds(c * LANES, LANES)]
            big_v[h, pl.ds(c * LANES, LANES)] = lax.shift_right_logical(v, 2)
            off_v[h, pl.ds(c * LANES, LANES)] = lax.shift_left(
                lax.bitwise_and(v, 3), 5)

    zeros = jnp.zeros((LANES,), jnp.float32)

    @pl.loop(0, HIDDEN)
    def _zero(j):
        for g in range(NGRP):
            accT[j, pl.ds(g * LANES, LANES)] = zeros

    pltpu.async_copy(in_emb_hbm.at[big_v.at[0]], rows0, sem0)
    pltpu.async_copy(in_emb_hbm.at[big_v.at[1]], rows1, sem1)

    lane_iota = lax.iota(jnp.int32, LANES)

    @pl.loop(0, HIST, step=2)
    def _h(h):
        for b, (rows, sem) in enumerate(((rows0, sem0), (rows1, sem1))):
            hc = h + b
            pltpu.make_async_copy(in_emb_hbm.at[big_v.at[hc]], rows, sem).wait()

            for g in range(NGRP):
                row_ids = lane_iota + (g * LANES)
                col0 = off_v[hc, pl.ds(g * LANES, LANES)]
                for j in range(HIDDEN):
                    val = plsc.load_gather(rows, [row_ids, col0 + j])
                    plsc.addupdate(accT.at[j, pl.ds(g * LANES, LANES)], val)

            @pl.when(hc + 2 < HIST)
            def _next():
                pltpu.async_copy(in_emb_hbm.at[big_v.at[hc + 2]], rows, sem)

    pltpu.sync_copy(accT, conT_hbm.at[:, pl.ds(base, BPW)])

    pltpu.make_async_copy(out_emb_hbm.at[tbig_v], rowst, semt).wait()
    for g in range(NGRP):
        row_ids = lane_iota + (g * LANES)
        col0 = toff_v[pl.ds(g * LANES, LANES)]
        for j in range(HIDDEN):
            val = plsc.load_gather(rowst, [row_ids, col0 + j])
            tgtT_v[j, pl.ds(g * LANES, LANES)] = val
    pltpu.sync_copy(tgtT_v, tgtT_hbm.at[:, pl.ds(base, BPW)])


def _sc_gather(ctxT, t_flat, in_emb4, out_emb4):
    mesh = plsc.VectorSubcoreMesh(core_axis_name="c", subcore_axis_name="s",
                                  num_cores=NC, num_subcores=NS)
    f = pl.kernel(
        _sc_body,
        out_type=(jax.ShapeDtypeStruct((HIDDEN, BATCH), jnp.float32),
                  jax.ShapeDtypeStruct((HIDDEN, BATCH), jnp.float32)),
        mesh=mesh,
        compiler_params=pltpu.CompilerParams(needs_layout_passes=False),
        scratch_types=[
            pltpu.VMEM((HIST, BPW), jnp.int32),
            pltpu.VMEM((BPW,), jnp.int32),
            pltpu.VMEM((HIST, BPW), jnp.int32),
            pltpu.VMEM((HIST, BPW), jnp.int32),
            pltpu.VMEM((BPW,), jnp.int32),
            pltpu.VMEM((BPW,), jnp.int32),
            pltpu.VMEM((BPW, 128), jnp.float32),
            pltpu.VMEM((BPW, 128), jnp.float32),
            pltpu.VMEM((BPW, 128), jnp.float32),
            pltpu.VMEM((HIDDEN, BPW), jnp.float32),
            pltpu.VMEM((HIDDEN, BPW), jnp.float32),
            pltpu.SemaphoreType.DMA,
            pltpu.SemaphoreType.DMA,
            pltpu.SemaphoreType.DMA,
        ],
    )
    return f(ctxT, t_flat, in_emb4, out_emb4)


def _mm_body(a_ref, b_ref, o_ref):
    o_ref[...] = lax.dot_general(a_ref[...], b_ref[...],
                                 (((0,), (0,)), ((), ())),
                                 preferred_element_type=jnp.float32)


def _tc_matmul(conT, tgtT):
    blk = 1024
    return pl.pallas_call(
        _mm_body,
        grid=(BATCH // blk, BATCH // blk),
        in_specs=[pl.BlockSpec((HIDDEN, blk), lambda i, j: (0, i)),
                  pl.BlockSpec((HIDDEN, blk), lambda i, j: (0, j))],
        out_specs=pl.BlockSpec((blk, blk), lambda i, j: (i, j)),
        out_shape=jax.ShapeDtypeStruct((BATCH, BATCH), jnp.float32),
    )(conT, tgtT)


def kernel(contexts, t, in_emb, out_emb):
    ctxT = contexts.T
    t_flat = t.reshape(BATCH)
    in_emb4 = in_emb.reshape(VOCAB // RPB, 128)
    out_emb4 = out_emb.reshape(VOCAB // RPB, 128)
    conT, tgtT = _sc_gather(ctxT, t_flat, in_emb4, out_emb4)
    return _tc_matmul(conT, tgtT)
